# baseline (speedup 1.0000x reference)
import functools

import jax
import jax.numpy as jnp
from jax import lax
from jax.experimental import pallas as pl
from jax.experimental.pallas import tpu as pltpu
from jax.experimental.pallas import tpu_sc as plsc

NUM_RELATION = 18
DIM = 32
MARGIN = 1.0
BATCH = 16384

NUM_CORES = 2
NUM_SUBCORES = 16
LANES = 16
NW = NUM_CORES * NUM_SUBCORES
B_PER_W = BATCH // NW
CHUNK = 128
N_CHUNKS = B_PER_W // CHUNK
GROUPS = B_PER_W // LANES


def _rsqrt(x):
    i = lax.bitcast_convert_type(x, jnp.int32)
    i = 0x5F3759DF - lax.shift_right_logical(i, 1)
    y = lax.bitcast_convert_type(i, jnp.float32)
    for _ in range(3):
        y = y * (1.5 - 0.5 * x * y * y)
    return y


def _safe_norm(sumsq):
    s = jnp.maximum(sumsq, 1e-24)
    return s * _rsqrt(s)


def _relu(x):
    return jnp.maximum(x, 0.0)


def _score(vd, tr, lb, rb, is_hypo, is_hyper, is_syn):
    hypo = _relu(vd - (lb - rb))
    hyper = _relu(vd - (rb - lb))
    syn = vd + jnp.abs(lb - rb)
    return jnp.where(is_hypo, hypo,
                     jnp.where(is_hyper, hyper,
                               jnp.where(is_syn, syn, tr)))


def _sc_body(left_hbm, right_hbm, rel_hbm, negl_hbm, negr_hbm,
             vec_hbm, biasg_hbm, relemb_hbm, out_hbm,
             li_v, ri_v, nli_v, nri_v, rel_v,
             l_v, r_v, nl_v, nr_v,
             lb_v, rb_v, nlb_v, nrb_v,
             relemb_v, acc_v, sem):
    wid = lax.axis_index("s") * NUM_CORES + lax.axis_index("c")
    base = wid * B_PER_W

    pltpu.sync_copy(left_hbm.at[pl.ds(base, B_PER_W)], li_v)
    pltpu.sync_copy(right_hbm.at[pl.ds(base, B_PER_W)], ri_v)
    pltpu.sync_copy(negl_hbm.at[pl.ds(base, B_PER_W)], nli_v)
    pltpu.sync_copy(negr_hbm.at[pl.ds(base, B_PER_W)], nri_v)
    pltpu.sync_copy(rel_hbm.at[pl.ds(base, B_PER_W)], rel_v)
    pltpu.sync_copy(relemb_hbm, relemb_v)
    pltpu.sync_copy(biasg_hbm.at[pl.ds(0 * BATCH + base, B_PER_W)], lb_v)
    pltpu.sync_copy(biasg_hbm.at[pl.ds(1 * BATCH + base, B_PER_W)], rb_v)
    pltpu.sync_copy(biasg_hbm.at[pl.ds(2 * BATCH + base, B_PER_W)], nlb_v)
    pltpu.sync_copy(biasg_hbm.at[pl.ds(3 * BATCH + base, B_PER_W)], nrb_v)

    copies = []
    for idx_v, vdst in ((li_v, l_v), (ri_v, r_v),
                        (nli_v, nl_v), (nri_v, nr_v)):
        for j in range(N_CHUNKS):
            sl = pl.ds(j * CHUNK, CHUNK)
            copies.append(pltpu.async_copy(
                vec_hbm.at[idx_v.at[sl]], vdst.at[sl, :], sem))
    for c in copies:
        c.wait()

    iota16 = lax.iota(jnp.int32, 16)
    zcol = jnp.zeros((LANES,), jnp.int32)
    zf = jnp.zeros((LANES,), jnp.float32)

    def group_body(g, cost_acc):
        rows = g * LANES + iota16
        rel_idx = rel_v[pl.ds(g * LANES, LANES)]
        lb = lb_v[pl.ds(g * LANES, LANES)]
        rb = rb_v[pl.ds(g * LANES, LANES)]
        nlb = nlb_v[pl.ds(g * LANES, LANES)]
        nrb = nrb_v[pl.ds(g * LANES, LANES)]

        def dim_body(d, accs):
            a1, a2, a3, a4, a5, a6 = accs
            dcol = jnp.full((LANES,), d, jnp.int32)
            l = plsc.load_gather(l_v, [rows, dcol])
            r = plsc.load_gather(r_v, [rows, dcol])
            nl = plsc.load_gather(nl_v, [rows, dcol])
            nr = plsc.load_gather(nr_v, [rows, dcol])
            s = plsc.load_gather(relemb_v, [rel_idx, dcol])
            t1 = l - r
            t2 = nl - r
            t3 = l - nr
            u1 = t1 + s
            u2 = t2 + s
            u3 = t3 + s
            return (a1 + t1 * t1, a2 + u1 * u1, a3 + t2 * t2,
                    a4 + u2 * u2, a5 + t3 * t3, a6 + u3 * u3)

        a1, a2, a3, a4, a5, a6 = lax.fori_loop(
            0, DIM, dim_body, (zf, zf, zf, zf, zf, zf))

        is_hypo = (rel_idx == 4) | (rel_idx == 6)
        is_hyper = (rel_idx == 3) | (rel_idx == 5)
        is_syn = ((rel_idx == 0) | (rel_idx == 1) |
                  (rel_idx == 13) | (rel_idx == 17))

        crt = _score(_safe_norm(a1), _safe_norm(a2), lb, rb,
                     is_hypo, is_hyper, is_syn)
        crtln = _score(_safe_norm(a3), _safe_norm(a4), nlb, rb,
                       is_hypo, is_hyper, is_syn)
        crtrn = _score(_safe_norm(a5), _safe_norm(a6), lb, nrb,
                       is_hypo, is_hyper, is_syn)
        cost = _relu(crt - crtln + MARGIN) + _relu(crt - crtrn + MARGIN)
        return cost_acc + cost

    cost_acc = lax.fori_loop(0, GROUPS, group_body, zf)
    acc_v[...] = cost_acc
    pltpu.sync_copy(acc_v, out_hbm.at[wid])


_sc_call = functools.partial(
    pl.kernel,
    out_type=jax.ShapeDtypeStruct((NW, LANES), jnp.float32),
    mesh=plsc.VectorSubcoreMesh(core_axis_name="c", subcore_axis_name="s"),
    compiler_params=pltpu.CompilerParams(
        needs_layout_passes=False, use_tc_tiling_on_sc=False),
    scratch_types=[
        pltpu.VMEM((B_PER_W,), jnp.int32),
        pltpu.VMEM((B_PER_W,), jnp.int32),
        pltpu.VMEM((B_PER_W,), jnp.int32),
        pltpu.VMEM((B_PER_W,), jnp.int32),
        pltpu.VMEM((B_PER_W,), jnp.int32),
        pltpu.VMEM((B_PER_W, DIM), jnp.float32),
        pltpu.VMEM((B_PER_W, DIM), jnp.float32),
        pltpu.VMEM((B_PER_W, DIM), jnp.float32),
        pltpu.VMEM((B_PER_W, DIM), jnp.float32),
        pltpu.VMEM((B_PER_W,), jnp.float32),
        pltpu.VMEM((B_PER_W,), jnp.float32),
        pltpu.VMEM((B_PER_W,), jnp.float32),
        pltpu.VMEM((B_PER_W,), jnp.float32),
        pltpu.VMEM((NUM_RELATION, DIM), jnp.float32),
        pltpu.VMEM((LANES,), jnp.float32),
        pltpu.SemaphoreType.DMA,
    ],
)(_sc_body)


@jax.jit
def kernel(leftEnIndices, rightEnIndices, relIndices, negLeftEnIndices,
           negRightEnIndices, predVec, predBias, relationEmbedding):
    li = leftEnIndices.astype(jnp.int32)
    ri = rightEnIndices.astype(jnp.int32)
    nli = negLeftEnIndices.astype(jnp.int32)
    nri = negRightEnIndices.astype(jnp.int32)
    all_idx = jnp.concatenate([li, ri, nli, nri])
    bias_g = jnp.take(predBias, all_idx, axis=0)[:, 0]
    partials = _sc_call(
        li, ri, relIndices.astype(jnp.int32), nli, nri,
        predVec, bias_g, relationEmbedding)
    return jnp.sum(partials) / BATCH


# speedup vs baseline: 1.4532x; 1.4532x over previous
import functools

import jax
import jax.numpy as jnp
from jax import lax
from jax.experimental import pallas as pl
from jax.experimental.pallas import tpu as pltpu
from jax.experimental.pallas import tpu_sc as plsc

NUM_RELATION = 18
DIM = 32
MARGIN = 1.0
BATCH = 16384

NUM_CORES = 2
NUM_SUBCORES = 16
LANES = 16
NW = NUM_CORES * NUM_SUBCORES
B_PER_W = BATCH // NW
CHUNK = 128
N_CHUNKS = B_PER_W // CHUNK
GROUPS = CHUNK // LANES


def _rsqrt(x):
    i = lax.bitcast_convert_type(x, jnp.int32)
    i = 0x5F3759DF - lax.shift_right_logical(i, 1)
    y = lax.bitcast_convert_type(i, jnp.float32)
    for _ in range(3):
        y = y * (1.5 - 0.5 * x * y * y)
    return y


def _safe_norm(sumsq):
    s = jnp.maximum(sumsq, 1e-24)
    return s * _rsqrt(s)


def _relu(x):
    return jnp.maximum(x, 0.0)


def _score(vd, tr, lb, rb, is_hypo, is_hyper, is_syn):
    hypo = _relu(vd - (lb - rb))
    hyper = _relu(vd - (rb - lb))
    syn = vd + jnp.abs(lb - rb)
    return jnp.where(is_hypo, hypo,
                     jnp.where(is_hyper, hyper,
                               jnp.where(is_syn, syn, tr)))


def _sc_body(left_hbm, right_hbm, rel_hbm, negl_hbm, negr_hbm,
             vec_hbm, biasg_hbm, relemb_hbm, out_hbm,
             li_v, ri_v, nli_v, nri_v, rel_v,
             l_v, r_v, nl_v, nr_v,
             lb_v, rb_v, nlb_v, nrb_v,
             relemb_v, acc_v, sem):
    wid = lax.axis_index("s") * NUM_CORES + lax.axis_index("c")
    base = wid * B_PER_W

    pltpu.sync_copy(left_hbm.at[pl.ds(base, B_PER_W)], li_v)
    pltpu.sync_copy(right_hbm.at[pl.ds(base, B_PER_W)], ri_v)
    pltpu.sync_copy(negl_hbm.at[pl.ds(base, B_PER_W)], nli_v)
    pltpu.sync_copy(negr_hbm.at[pl.ds(base, B_PER_W)], nri_v)
    pltpu.sync_copy(rel_hbm.at[pl.ds(base, B_PER_W)], rel_v)
    pltpu.sync_copy(biasg_hbm.at[pl.ds(0 * BATCH + base, B_PER_W)], lb_v)
    pltpu.sync_copy(biasg_hbm.at[pl.ds(1 * BATCH + base, B_PER_W)], rb_v)
    pltpu.sync_copy(biasg_hbm.at[pl.ds(2 * BATCH + base, B_PER_W)], nlb_v)
    pltpu.sync_copy(biasg_hbm.at[pl.ds(3 * BATCH + base, B_PER_W)], nrb_v)
    for rr in range(NUM_RELATION):
        pltpu.async_copy(relemb_hbm.at[pl.ds(rr, 1), :],
                         relemb_v.at[pl.ds(rr, 1), :], sem)
    pltpu.make_async_copy(relemb_hbm, relemb_v, sem).wait()

    iota16 = lax.iota(jnp.int32, 16)
    zf = jnp.zeros((LANES,), jnp.float32)

    def chunk_body(c, total):
        cbase = c * CHUNK

        for idx_ref, vdst in ((li_v, l_v), (ri_v, r_v),
                              (nli_v, nl_v), (nri_v, nr_v)):
            def enq(g, carry, idx_ref=idx_ref, vdst=vdst):
                vec = idx_ref[pl.ds(cbase + g * LANES, LANES)]
                for lane in range(LANES):
                    i = vec[lane]
                    e = g * LANES + lane
                    pltpu.async_copy(vec_hbm.at[pl.ds(i, 1), :],
                                     vdst.at[pl.ds(e, 1), :], sem)
                return carry
            lax.fori_loop(0, GROUPS, enq, 0)
        for vdst in (l_v, r_v, nl_v, nr_v):
            pltpu.make_async_copy(vec_hbm.at[pl.ds(0, CHUNK), :], vdst,
                                  sem).wait()

        def group_body(g, cost_acc):
            rows = g * LANES + iota16
            gbase = cbase + g * LANES
            rel_idx = rel_v[pl.ds(gbase, LANES)]
            lb = lb_v[pl.ds(gbase, LANES)]
            rb = rb_v[pl.ds(gbase, LANES)]
            nlb = nlb_v[pl.ds(gbase, LANES)]
            nrb = nrb_v[pl.ds(gbase, LANES)]

            def dim_body(d, accs):
                a1, a2, a3, a4, a5, a6 = accs
                dcol = jnp.full((LANES,), d, jnp.int32)
                l = plsc.load_gather(l_v, [rows, dcol])
                r = plsc.load_gather(r_v, [rows, dcol])
                nl = plsc.load_gather(nl_v, [rows, dcol])
                nr = plsc.load_gather(nr_v, [rows, dcol])
                s = plsc.load_gather(relemb_v, [rel_idx, dcol])
                t1 = l - r
                t2 = nl - r
                t3 = l - nr
                u1 = t1 + s
                u2 = t2 + s
                u3 = t3 + s
                return (a1 + t1 * t1, a2 + u1 * u1, a3 + t2 * t2,
                        a4 + u2 * u2, a5 + t3 * t3, a6 + u3 * u3)

            a1, a2, a3, a4, a5, a6 = lax.fori_loop(
                0, DIM, dim_body, (zf, zf, zf, zf, zf, zf))

            is_hypo = (rel_idx == 4) | (rel_idx == 6)
            is_hyper = (rel_idx == 3) | (rel_idx == 5)
            is_syn = ((rel_idx == 0) | (rel_idx == 1) |
                      (rel_idx == 13) | (rel_idx == 17))

            crt = _score(_safe_norm(a1), _safe_norm(a2), lb, rb,
                         is_hypo, is_hyper, is_syn)
            crtln = _score(_safe_norm(a3), _safe_norm(a4), nlb, rb,
                           is_hypo, is_hyper, is_syn)
            crtrn = _score(_safe_norm(a5), _safe_norm(a6), lb, nrb,
                           is_hypo, is_hyper, is_syn)
            cost = _relu(crt - crtln + MARGIN) + _relu(crt - crtrn + MARGIN)
            return cost_acc + cost

        return lax.fori_loop(0, GROUPS, group_body, total)

    cost_acc = lax.fori_loop(0, N_CHUNKS, chunk_body, zf)
    acc_v[...] = cost_acc
    pltpu.sync_copy(acc_v, out_hbm.at[pl.ds(wid * LANES, LANES)])


_sc_call = functools.partial(
    pl.kernel,
    out_type=jax.ShapeDtypeStruct((NW * LANES,), jnp.float32),
    mesh=plsc.VectorSubcoreMesh(core_axis_name="c", subcore_axis_name="s"),
    compiler_params=pltpu.CompilerParams(needs_layout_passes=False),
    scratch_types=[
        pltpu.VMEM((B_PER_W,), jnp.int32),
        pltpu.VMEM((B_PER_W,), jnp.int32),
        pltpu.VMEM((B_PER_W,), jnp.int32),
        pltpu.VMEM((B_PER_W,), jnp.int32),
        pltpu.VMEM((B_PER_W,), jnp.int32),
        pltpu.VMEM((CHUNK, DIM), jnp.float32),
        pltpu.VMEM((CHUNK, DIM), jnp.float32),
        pltpu.VMEM((CHUNK, DIM), jnp.float32),
        pltpu.VMEM((CHUNK, DIM), jnp.float32),
        pltpu.VMEM((B_PER_W,), jnp.float32),
        pltpu.VMEM((B_PER_W,), jnp.float32),
        pltpu.VMEM((B_PER_W,), jnp.float32),
        pltpu.VMEM((B_PER_W,), jnp.float32),
        pltpu.VMEM((NUM_RELATION, DIM), jnp.float32),
        pltpu.VMEM((LANES,), jnp.float32),
        pltpu.SemaphoreType.DMA,
    ],
)(_sc_body)


@jax.jit
def kernel(leftEnIndices, rightEnIndices, relIndices, negLeftEnIndices,
           negRightEnIndices, predVec, predBias, relationEmbedding):
    li = leftEnIndices.astype(jnp.int32)
    ri = rightEnIndices.astype(jnp.int32)
    nli = negLeftEnIndices.astype(jnp.int32)
    nri = negRightEnIndices.astype(jnp.int32)
    all_idx = jnp.concatenate([li, ri, nli, nri])
    bias_g = jnp.take(predBias, all_idx, axis=0)[:, 0]
    partials = _sc_call(
        li, ri, relIndices.astype(jnp.int32), nli, nri,
        predVec, bias_g, relationEmbedding)
    return jnp.sum(partials) / BATCH


# speedup vs baseline: 1.8326x; 1.2611x over previous
import functools

import jax
import jax.numpy as jnp
from jax import lax
from jax.experimental import pallas as pl
from jax.experimental.pallas import tpu as pltpu
from jax.experimental.pallas import tpu_sc as plsc

NUM_RELATION = 18
DIM = 32
MARGIN = 1.0
BATCH = 16384

NUM_CORES = 2
NUM_SUBCORES = 16
LANES = 16
NW = NUM_CORES * NUM_SUBCORES
B_PER_W = BATCH // NW
CHUNK = 128
N_CHUNKS = B_PER_W // CHUNK
GROUPS = CHUNK // LANES


def _rsqrt(x):
    i = lax.bitcast_convert_type(x, jnp.int32)
    i = 0x5F3759DF - lax.shift_right_logical(i, 1)
    y = lax.bitcast_convert_type(i, jnp.float32)
    for _ in range(3):
        y = y * (1.5 - 0.5 * x * y * y)
    return y


def _safe_norm(sumsq):
    s = jnp.maximum(sumsq, 1e-24)
    return s * _rsqrt(s)


def _relu(x):
    return jnp.maximum(x, 0.0)


def _score(vd, tr, lb, rb, is_hypo, is_hyper, is_syn):
    hypo = _relu(vd - (lb - rb))
    hyper = _relu(vd - (rb - lb))
    syn = vd + jnp.abs(lb - rb)
    return jnp.where(is_hypo, hypo,
                     jnp.where(is_hyper, hyper,
                               jnp.where(is_syn, syn, tr)))


def _sc_body(left_hbm, right_hbm, rel_hbm, negl_hbm, negr_hbm,
             vec_hbm, biasg_hbm, relemb_hbm, out_hbm,
             li_v, ri_v, nli_v, nri_v, rel_v,
             l_v, r_v, nl_v, nr_v,
             lb_v, rb_v, nlb_v, nrb_v,
             relemb_v, acc_v, sem):
    wid = lax.axis_index("s") * NUM_CORES + lax.axis_index("c")
    base = wid * B_PER_W

    pltpu.sync_copy(left_hbm.at[pl.ds(base, B_PER_W)], li_v)
    pltpu.sync_copy(right_hbm.at[pl.ds(base, B_PER_W)], ri_v)
    pltpu.sync_copy(negl_hbm.at[pl.ds(base, B_PER_W)], nli_v)
    pltpu.sync_copy(negr_hbm.at[pl.ds(base, B_PER_W)], nri_v)
    pltpu.sync_copy(rel_hbm.at[pl.ds(base, B_PER_W)], rel_v)
    pltpu.sync_copy(biasg_hbm.at[pl.ds(0 * BATCH + base, B_PER_W)], lb_v)
    pltpu.sync_copy(biasg_hbm.at[pl.ds(1 * BATCH + base, B_PER_W)], rb_v)
    pltpu.sync_copy(biasg_hbm.at[pl.ds(2 * BATCH + base, B_PER_W)], nlb_v)
    pltpu.sync_copy(biasg_hbm.at[pl.ds(3 * BATCH + base, B_PER_W)], nrb_v)
    for rr in range(NUM_RELATION):
        pltpu.async_copy(relemb_hbm.at[pl.ds(rr, 1), :],
                         relemb_v.at[pl.ds(rr, 1), :], sem)
    pltpu.make_async_copy(relemb_hbm, relemb_v, sem).wait()

    iota16 = lax.iota(jnp.int32, 16)
    zf = jnp.zeros((LANES,), jnp.float32)

    def chunk_body(c, total):
        cbase = c * CHUNK

        for idx_ref, vdst in ((li_v, l_v), (ri_v, r_v),
                              (nli_v, nl_v), (nri_v, nr_v)):
            def enq(g, carry, idx_ref=idx_ref, vdst=vdst):
                vec = idx_ref[pl.ds(cbase + g * LANES, LANES)]
                for lane in range(LANES):
                    i = vec[lane]
                    e = g * LANES + lane
                    pltpu.async_copy(vec_hbm.at[pl.ds(i, 1), :],
                                     vdst.at[pl.ds(e, 1), :], sem)
                return carry
            lax.fori_loop(0, GROUPS, enq, 0)
        for vdst in (l_v, r_v, nl_v, nr_v):
            pltpu.make_async_copy(vec_hbm.at[pl.ds(0, CHUNK), :], vdst,
                                  sem).wait()

        def group_body(g, cost_acc):
            rows = g * LANES + iota16
            gbase = cbase + g * LANES
            rel_idx = rel_v[pl.ds(gbase, LANES)]
            lb = lb_v[pl.ds(gbase, LANES)]
            rb = rb_v[pl.ds(gbase, LANES)]
            nlb = nlb_v[pl.ds(gbase, LANES)]
            nrb = nrb_v[pl.ds(gbase, LANES)]

            def dim_body(d, accs):
                a1, a2, a3, a4, a5, a6 = accs
                dcol = jnp.full((LANES,), d, jnp.int32)
                l = plsc.load_gather(l_v, [rows, dcol])
                r = plsc.load_gather(r_v, [rows, dcol])
                nl = plsc.load_gather(nl_v, [rows, dcol])
                nr = plsc.load_gather(nr_v, [rows, dcol])
                s = plsc.load_gather(relemb_v, [rel_idx, dcol])
                t1 = l - r
                t2 = nl - r
                t3 = l - nr
                u1 = t1 + s
                u2 = t2 + s
                u3 = t3 + s
                return (a1 + t1 * t1, a2 + u1 * u1, a3 + t2 * t2,
                        a4 + u2 * u2, a5 + t3 * t3, a6 + u3 * u3)

            a1, a2, a3, a4, a5, a6 = lax.fori_loop(
                0, DIM, dim_body, (zf, zf, zf, zf, zf, zf))

            is_hypo = (rel_idx == 4) | (rel_idx == 6)
            is_hyper = (rel_idx == 3) | (rel_idx == 5)
            is_syn = ((rel_idx == 0) | (rel_idx == 1) |
                      (rel_idx == 13) | (rel_idx == 17))

            crt = _score(_safe_norm(a1), _safe_norm(a2), lb, rb,
                         is_hypo, is_hyper, is_syn)
            crtln = _score(_safe_norm(a3), _safe_norm(a4), nlb, rb,
                           is_hypo, is_hyper, is_syn)
            crtrn = _score(_safe_norm(a5), _safe_norm(a6), lb, nrb,
                           is_hypo, is_hyper, is_syn)
            cost = _relu(crt - crtln + MARGIN) + _relu(crt - crtrn + MARGIN)
            return cost_acc + cost

        return lax.fori_loop(0, GROUPS, group_body, total)

    cost_acc = lax.fori_loop(0, N_CHUNKS, chunk_body, zf)
    acc_v[...] = cost_acc
    pltpu.sync_copy(acc_v, out_hbm.at[pl.ds(wid * LANES, LANES)])


_sc_call = functools.partial(
    pl.kernel,
    out_type=jax.ShapeDtypeStruct((NW * LANES,), jnp.float32),
    mesh=plsc.VectorSubcoreMesh(core_axis_name="c", subcore_axis_name="s"),
    compiler_params=pltpu.CompilerParams(needs_layout_passes=False),
    scratch_types=[
        pltpu.VMEM((B_PER_W,), jnp.int32),
        pltpu.VMEM((B_PER_W,), jnp.int32),
        pltpu.VMEM((B_PER_W,), jnp.int32),
        pltpu.VMEM((B_PER_W,), jnp.int32),
        pltpu.VMEM((B_PER_W,), jnp.int32),
        pltpu.VMEM((CHUNK, DIM), jnp.float32),
        pltpu.VMEM((CHUNK, DIM), jnp.float32),
        pltpu.VMEM((CHUNK, DIM), jnp.float32),
        pltpu.VMEM((CHUNK, DIM), jnp.float32),
        pltpu.VMEM((B_PER_W,), jnp.float32),
        pltpu.VMEM((B_PER_W,), jnp.float32),
        pltpu.VMEM((B_PER_W,), jnp.float32),
        pltpu.VMEM((B_PER_W,), jnp.float32),
        pltpu.VMEM((NUM_RELATION, DIM), jnp.float32),
        pltpu.VMEM((LANES,), jnp.float32),
        pltpu.SemaphoreType.DMA,
    ],
)(_sc_body)


@jax.jit
def kernel(leftEnIndices, rightEnIndices, relIndices, negLeftEnIndices,
           negRightEnIndices, predVec, predBias, relationEmbedding):
    li = leftEnIndices.astype(jnp.int32)
    ri = rightEnIndices.astype(jnp.int32)
    nli = negLeftEnIndices.astype(jnp.int32)
    nri = negRightEnIndices.astype(jnp.int32)
    relayout = (jnp.eye(DIM, dtype=jnp.float32)
                + jnp.full((DIM, DIM), 1e-30, jnp.float32))
    vec_rm = predVec @ relayout
    all_idx = jnp.concatenate([li, ri, nli, nri])
    bias_g = jnp.take(predBias, all_idx, axis=0)[:, 0]
    partials = _sc_call(
        li, ri, relIndices.astype(jnp.int32), nli, nri,
        vec_rm, bias_g, relationEmbedding)
    return jnp.sum(partials) / BATCH
